# baseline (speedup 1.0000x reference)
import functools

import jax
import jax.numpy as jnp
from jax import lax
from jax.experimental import pallas as pl
from jax.experimental.pallas import tpu as pltpu
from jax.experimental.pallas import tpu_sc as plsc

_VOCAB = 1000000
_EMB = 32
_B = 16384
_L = 200
_PAD = 0

_PACK = 128 // _EMB
_ROWS_A = 5000
_GRID_A = (_VOCAB // _PACK) // _ROWS_A

_NC, _NS = 2, 16
_NW = _NC * _NS
_COLS_W = _B // _NW
_CH = 256
_NCH = _COLS_W // _CH
_CHW = _L * _CH


def _score_body(x_ref, w_ref, out_ref):
    x = x_ref[...]
    w = w_ref[...]
    s = lax.dot_general(x, w, (((1,), (0,)), ((), ())),
                        preferred_element_type=jnp.float32)
    i = pl.program_id(0)
    row = lax.broadcasted_iota(jnp.int32, (_ROWS_A, _PACK), 0)
    col = lax.broadcasted_iota(jnp.int32, (_ROWS_A, _PACK), 1)
    s = jnp.where((i == 0) & (row == 0) & (col == _PAD), 0.0, s)
    out_ref[...] = s


def _scores(table, W):
    x = table.reshape(_VOCAB // _PACK, 128)
    w = W.reshape(_EMB)
    w4 = (jnp.tile(w, _PACK)[:, None]
          * jnp.repeat(jnp.eye(_PACK, dtype=jnp.float32), _EMB, axis=0)
          * (1.0 / _L))
    out = pl.pallas_call(
        _score_body,
        grid=(_GRID_A,),
        in_specs=[
            pl.BlockSpec((_ROWS_A, 128), lambda i: (i, 0)),
            pl.BlockSpec((128, _PACK), lambda i: (0, 0)),
        ],
        out_specs=pl.BlockSpec((_ROWS_A, _PACK), lambda i: (i, 0)),
        out_shape=jax.ShapeDtypeStruct((_VOCAB // _PACK, _PACK), jnp.float32),
    )(x, w4)
    return out.reshape(_VOCAB)


def _tr_body(ids_ref, out_ref):
    out_ref[...] = ids_ref[...].T


def _transpose_ids(ids):
    return pl.pallas_call(
        _tr_body,
        grid=(_B // 128,),
        in_specs=[pl.BlockSpec((128, _L), lambda c: (c, 0))],
        out_specs=pl.BlockSpec((_L, 128), lambda c: (0, c)),
        out_shape=jax.ShapeDtypeStruct((_L, _B), jnp.int32),
    )(ids)


def _sc_pool_body(scores_hbm, ids_hbm, bvec_hbm, out_hbm,
                  idx_v, vals_v, out_v, b_v, sem, sem2):
    wid = lax.axis_index("s") * _NC + lax.axis_index("c")
    base = wid * _COLS_W
    pltpu.sync_copy(bvec_hbm, b_v)
    bv = b_v[...]

    def chunk(ci, carry):
        col0 = base + ci * _CH

        def cp_issue(l, c2):
            pltpu.async_copy(ids_hbm.at[l, pl.ds(col0, _CH)],
                             idx_v.at[pl.ds(l * _CH, _CH)], sem2)
            return c2

        def cp_drain(l, c2):
            pltpu.make_async_copy(ids_hbm.at[l, pl.ds(col0, _CH)],
                                  idx_v.at[pl.ds(l * _CH, _CH)], sem2).wait()
            return c2

        lax.fori_loop(0, _L, cp_issue, 0)
        lax.fori_loop(0, _L, cp_drain, 0)
        pltpu.async_copy(scores_hbm.at[idx_v], vals_v, sem).wait()

        for k in range(_CH // 16):
            def red(l, acc):
                return acc + vals_v[pl.ds(l * _CH + k * 16, 16)]
            acc = lax.fori_loop(0, _L, red, jnp.zeros((16,), jnp.float32))
            z = acc + bv
            y = 1.0 / (1.0 + jnp.exp(-z))
            out_v[pl.ds(ci * _CH + k * 16, 16)] = y
        return carry

    lax.fori_loop(0, _NCH, chunk, 0)
    pltpu.sync_copy(out_v, out_hbm.at[pl.ds(base, _COLS_W)])


def _sc_pool(scores, ids_t, bvec):
    mesh = plsc.VectorSubcoreMesh(core_axis_name="c", subcore_axis_name="s")
    f = pl.kernel(
        _sc_pool_body,
        out_type=jax.ShapeDtypeStruct((_B,), jnp.float32),
        mesh=mesh,
        scratch_types=[
            pltpu.VMEM((_CHW,), jnp.int32),
            pltpu.VMEM((_CHW,), jnp.float32),
            pltpu.VMEM((_COLS_W,), jnp.float32),
            pltpu.VMEM((16,), jnp.float32),
            pltpu.SemaphoreType.DMA,
            pltpu.SemaphoreType.DMA,
        ],
    )
    return f(scores, ids_t, bvec)


def kernel(ids, table, W, b):
    scores = _scores(table.astype(jnp.float32), W.astype(jnp.float32))
    ids_t = _transpose_ids(ids.astype(jnp.int32))
    bvec = jnp.broadcast_to(b.astype(jnp.float32), (16,))
    out_flat = _sc_pool(scores, ids_t, bvec)
    return out_flat.reshape(_B, 1)


# speedup vs baseline: 3.2164x; 3.2164x over previous
import functools

import jax
import jax.numpy as jnp
from jax import lax
from jax.experimental import pallas as pl
from jax.experimental.pallas import tpu as pltpu
from jax.experimental.pallas import tpu_sc as plsc

_VOCAB = 1000000
_EMB = 32
_B = 16384
_L = 200
_PAD = 0

_COLS_A = 8192
_GRID_A = -(-_VOCAB // _COLS_A)

_NC, _NS = 2, 16
_NW = _NC * _NS
_COLS_W = _B // _NW
_CH = 256
_NCH = _COLS_W // _CH
_CHW = _L * _CH


def _score_body(x_ref, w_ref, out_ref):
    x = x_ref[...]
    w = w_ref[...]
    s = jnp.sum(x * w, axis=0, keepdims=True)
    i = pl.program_id(0)
    col = lax.broadcasted_iota(jnp.int32, (1, _COLS_A), 1)
    s = jnp.where((i == 0) & (col == _PAD), 0.0, s)
    out_ref[...] = s.reshape(_COLS_A)


def _scores(table_t, w_col):
    return pl.pallas_call(
        _score_body,
        grid=(_GRID_A,),
        in_specs=[
            pl.BlockSpec((_EMB, _COLS_A), lambda i: (0, i)),
            pl.BlockSpec((_EMB, 1), lambda i: (0, 0)),
        ],
        out_specs=pl.BlockSpec((_COLS_A,), lambda i: (i,)),
        out_shape=jax.ShapeDtypeStruct((_VOCAB,), jnp.float32),
    )(table_t, w_col)


def _sc_pool_body(scores_hbm, ids_hbm, bvec_hbm, out_hbm,
                  idx_v, vals_v, out_v, b_v, sem, sem2):
    wid = lax.axis_index("s") * _NC + lax.axis_index("c")
    base = wid * _COLS_W
    pltpu.sync_copy(bvec_hbm, b_v)
    bv = b_v[...]

    def chunk(ci, carry):
        col0 = base + ci * _CH

        def cp_issue(l, c2):
            pltpu.async_copy(ids_hbm.at[l, pl.ds(col0, _CH)],
                             idx_v.at[pl.ds(l * _CH, _CH)], sem2)
            return c2

        def cp_drain(l, c2):
            pltpu.make_async_copy(ids_hbm.at[l, pl.ds(col0, _CH)],
                                  idx_v.at[pl.ds(l * _CH, _CH)], sem2).wait()
            return c2

        lax.fori_loop(0, _L, cp_issue, 0)
        lax.fori_loop(0, _L, cp_drain, 0)
        pltpu.async_copy(scores_hbm.at[idx_v], vals_v, sem).wait()

        for k in range(_CH // 16):
            def red(l, acc):
                return acc + vals_v[pl.ds(l * _CH + k * 16, 16)]
            acc = lax.fori_loop(0, _L, red, jnp.zeros((16,), jnp.float32))
            z = acc + bv
            y = 1.0 / (1.0 + jnp.exp(-z))
            out_v[pl.ds(ci * _CH + k * 16, 16)] = y
        return carry

    lax.fori_loop(0, _NCH, chunk, 0)
    pltpu.sync_copy(out_v, out_hbm.at[pl.ds(base, _COLS_W)])


def _sc_pool(scores, ids_t, bvec):
    mesh = plsc.VectorSubcoreMesh(core_axis_name="c", subcore_axis_name="s")
    f = pl.kernel(
        _sc_pool_body,
        out_type=jax.ShapeDtypeStruct((_B,), jnp.float32),
        mesh=mesh,
        scratch_types=[
            pltpu.VMEM((_CHW,), jnp.int32),
            pltpu.VMEM((_CHW,), jnp.float32),
            pltpu.VMEM((_COLS_W,), jnp.float32),
            pltpu.VMEM((16,), jnp.float32),
            pltpu.SemaphoreType.DMA,
            pltpu.SemaphoreType.DMA,
        ],
    )
    return f(scores, ids_t, bvec)


def kernel(ids, table, W, b):
    table_t = table.astype(jnp.float32).T
    ids_t = ids.astype(jnp.int32).T
    w_col = W.astype(jnp.float32).reshape(_EMB, 1) * (1.0 / _L)
    scores = _scores(table_t, w_col)
    bvec = jnp.broadcast_to(b.astype(jnp.float32), (16,))
    out_flat = _sc_pool(scores, ids_t, bvec)
    return out_flat.reshape(_B, 1)


# speedup vs baseline: 3.4448x; 1.0710x over previous
import functools

import jax
import jax.numpy as jnp
from jax import lax
from jax.experimental import pallas as pl
from jax.experimental.pallas import tpu as pltpu
from jax.experimental.pallas import tpu_sc as plsc

_VOCAB = 1000000
_EMB = 32
_B = 16384
_L = 200
_PAD = 0

_COLS_A = 8192
_GRID_A = -(-_VOCAB // _COLS_A)

_NC, _NS = 2, 16
_NW = _NC * _NS
_COLS_W = _B // _NW
_CH = 128
_NCH = _COLS_W // _CH
_CHW = _L * _CH


def _score_body(x_ref, w_ref, out_ref):
    x = x_ref[...]
    w = w_ref[...]
    s = jnp.sum(x * w, axis=0, keepdims=True)
    i = pl.program_id(0)
    col = lax.broadcasted_iota(jnp.int32, (1, _COLS_A), 1)
    s = jnp.where((i == 0) & (col == _PAD), 0.0, s)
    out_ref[...] = s.reshape(_COLS_A)


def _scores(table_t, w_col):
    return pl.pallas_call(
        _score_body,
        grid=(_GRID_A,),
        in_specs=[
            pl.BlockSpec((_EMB, _COLS_A), lambda i: (0, i)),
            pl.BlockSpec((_EMB, 1), lambda i: (0, 0)),
        ],
        out_specs=pl.BlockSpec((_COLS_A,), lambda i: (i,)),
        out_shape=jax.ShapeDtypeStruct((_VOCAB,), jnp.float32),
    )(table_t, w_col)


def _sc_pool_body(scores_hbm, ids_hbm, bvec_hbm, out_hbm,
                  idx0, idx1, vals0, vals1, out_v, b_v, semg, sema):
    wid = lax.axis_index("s") * _NC + lax.axis_index("c")
    base = wid * _COLS_W
    pltpu.sync_copy(bvec_hbm, b_v)
    bv = b_v[...]
    idx = [idx0, idx1]
    vals = [vals0, vals1]

    def assemble(ci, buf):
        col0 = base + ci * _CH

        def cp_issue(l, c2):
            pltpu.async_copy(ids_hbm.at[l, pl.ds(col0, _CH)],
                             buf.at[pl.ds(l * _CH, _CH)], sema)
            return c2

        def cp_drain(l, c2):
            pltpu.make_async_copy(ids_hbm.at[l, pl.ds(col0, _CH)],
                                  buf.at[pl.ds(l * _CH, _CH)], sema).wait()
            return c2

        lax.fori_loop(0, _L, cp_issue, 0)
        lax.fori_loop(0, _L, cp_drain, 0)

    assemble(0, idx[0])
    pltpu.async_copy(scores_hbm.at[idx[0]], vals[0], semg)
    assemble(1, idx[1])
    for ci in range(_NCH):
        cur = ci % 2
        pltpu.make_async_copy(scores_hbm.at[idx[cur]], vals[cur], semg).wait()
        if ci + 1 < _NCH:
            pltpu.async_copy(scores_hbm.at[idx[1 - cur]], vals[1 - cur], semg)
        if ci + 2 < _NCH:
            assemble(ci + 2, idx[cur])

        for k in range(_CH // 16):
            def red(l, acc):
                return acc + vals[cur][pl.ds(l * _CH + k * 16, 16)]
            acc = lax.fori_loop(0, _L, red, jnp.zeros((16,), jnp.float32))
            z = acc + bv
            y = 1.0 / (1.0 + jnp.exp(-z))
            out_v[pl.ds(ci * _CH + k * 16, 16)] = y

    pltpu.sync_copy(out_v, out_hbm.at[pl.ds(base, _COLS_W)])


def _sc_pool(scores, ids_t, bvec):
    mesh = plsc.VectorSubcoreMesh(core_axis_name="c", subcore_axis_name="s")
    f = pl.kernel(
        _sc_pool_body,
        out_type=jax.ShapeDtypeStruct((_B,), jnp.float32),
        mesh=mesh,
        scratch_types=[
            pltpu.VMEM((_CHW,), jnp.int32),
            pltpu.VMEM((_CHW,), jnp.int32),
            pltpu.VMEM((_CHW,), jnp.float32),
            pltpu.VMEM((_CHW,), jnp.float32),
            pltpu.VMEM((_COLS_W,), jnp.float32),
            pltpu.VMEM((16,), jnp.float32),
            pltpu.SemaphoreType.DMA,
            pltpu.SemaphoreType.DMA,
        ],
    )
    return f(scores, ids_t, bvec)


def kernel(ids, table, W, b):
    table_t = table.astype(jnp.float32).T
    ids_t = ids.astype(jnp.int32).T
    w_col = W.astype(jnp.float32).reshape(_EMB, 1) * (1.0 / _L)
    scores = _scores(table_t, w_col)
    bvec = jnp.broadcast_to(b.astype(jnp.float32), (16,))
    out_flat = _sc_pool(scores, ids_t, bvec)
    return out_flat.reshape(_B, 1)


# speedup vs baseline: 3.9021x; 1.1328x over previous
import functools

import jax
import jax.numpy as jnp
from jax import lax
from jax.experimental import pallas as pl
from jax.experimental.pallas import tpu as pltpu
from jax.experimental.pallas import tpu_sc as plsc

_VOCAB = 1000000
_EMB = 32
_B = 16384
_L = 200
_PAD = 0

_COLS_A = 16384
_GRID_A = -(-_VOCAB // _COLS_A)

_NC, _NS = 2, 16
_NW = _NC * _NS
_COLS_W = _B // _NW
_CH = 128
_NCH = _COLS_W // _CH
_CHW = _L * _CH


def _score_body(x_ref, w_ref, out_ref):
    x = x_ref[...]
    w = w_ref[...]
    s = jnp.sum(x * w, axis=0, keepdims=True)
    i = pl.program_id(0)
    col = lax.broadcasted_iota(jnp.int32, (1, _COLS_A), 1)
    s = jnp.where((i == 0) & (col == _PAD), 0.0, s)
    out_ref[...] = s.reshape(_COLS_A)


def _scores(table_t, w_col):
    return pl.pallas_call(
        _score_body,
        grid=(_GRID_A,),
        in_specs=[
            pl.BlockSpec((_EMB, _COLS_A), lambda i: (0, i)),
            pl.BlockSpec((_EMB, 1), lambda i: (0, 0)),
        ],
        out_specs=pl.BlockSpec((_COLS_A,), lambda i: (i,)),
        out_shape=jax.ShapeDtypeStruct((_VOCAB,), jnp.float32),
    )(table_t, w_col)


def _sc_pool_body(scores_hbm, ids_hbm, bvec_hbm, out_hbm,
                  idx0, idx1, vals0, vals1, out_v, b_v, semg, sema):
    wid = lax.axis_index("s") * _NC + lax.axis_index("c")
    base = wid * _COLS_W
    pltpu.sync_copy(bvec_hbm, b_v)
    bv = b_v[...]
    idx = [idx0, idx1]
    vals = [vals0, vals1]

    def assemble(ci, buf):
        col0 = base + ci * _CH

        def cp_issue(l, c2):
            pltpu.async_copy(ids_hbm.at[l, pl.ds(col0, _CH)],
                             buf.at[pl.ds(l * _CH, _CH)], sema)
            return c2

        def cp_drain(l, c2):
            pltpu.make_async_copy(ids_hbm.at[l, pl.ds(col0, _CH)],
                                  buf.at[pl.ds(l * _CH, _CH)], sema).wait()
            return c2

        lax.fori_loop(0, _L, cp_issue, 0)
        lax.fori_loop(0, _L, cp_drain, 0)

    assemble(0, idx[0])
    pltpu.async_copy(scores_hbm.at[idx[0]], vals[0], semg)
    assemble(1, idx[1])
    for ci in range(_NCH):
        cur = ci % 2
        pltpu.make_async_copy(scores_hbm.at[idx[cur]], vals[cur], semg).wait()
        if ci + 1 < _NCH:
            pltpu.async_copy(scores_hbm.at[idx[1 - cur]], vals[1 - cur], semg)
        if ci + 2 < _NCH:
            assemble(ci + 2, idx[cur])

        for k in range(_CH // 16):
            def red(l, acc):
                return acc + vals[cur][pl.ds(l * _CH + k * 16, 16)]
            acc = lax.fori_loop(0, _L, red, jnp.zeros((16,), jnp.float32))
            z = acc + bv
            y = 1.0 / (1.0 + jnp.exp(-z))
            out_v[pl.ds(ci * _CH + k * 16, 16)] = y

    pltpu.sync_copy(out_v, out_hbm.at[pl.ds(base, _COLS_W)])


def _sc_pool(scores, ids_t, bvec):
    mesh = plsc.VectorSubcoreMesh(core_axis_name="c", subcore_axis_name="s")
    f = pl.kernel(
        _sc_pool_body,
        out_type=jax.ShapeDtypeStruct((_B,), jnp.float32),
        mesh=mesh,
        scratch_types=[
            pltpu.VMEM((_CHW,), jnp.int32),
            pltpu.VMEM((_CHW,), jnp.int32),
            pltpu.VMEM((_CHW,), jnp.float32),
            pltpu.VMEM((_CHW,), jnp.float32),
            pltpu.VMEM((_COLS_W,), jnp.float32),
            pltpu.VMEM((16,), jnp.float32),
            pltpu.SemaphoreType.DMA,
            pltpu.SemaphoreType.DMA,
        ],
    )
    return f(scores, ids_t, bvec)


def kernel(ids, table, W, b):
    table_t = table.astype(jnp.float32).T
    ids_t = ids.astype(jnp.int32).T
    w_col = W.astype(jnp.float32).reshape(_EMB, 1) * (1.0 / _L)
    scores = _scores(table_t, w_col)
    bvec = jnp.broadcast_to(b.astype(jnp.float32), (16,))
    out_flat = _sc_pool(scores, ids_t, bvec)
    return out_flat.reshape(_B, 1)


# speedup vs baseline: 4.2305x; 1.0841x over previous
import functools

import jax
import jax.numpy as jnp
from jax import lax
from jax.experimental import pallas as pl
from jax.experimental.pallas import tpu as pltpu
from jax.experimental.pallas import tpu_sc as plsc

_VOCAB = 1000000
_EMB = 32
_B = 16384
_L = 200
_PAD = 0

_COLS_A = 32768
_GRID_A = -(-_VOCAB // _COLS_A)

_NC, _NS = 2, 16
_NW = _NC * _NS
_COLS_W = _B // _NW
_CH = 128
_NCH = _COLS_W // _CH
_CHW = _L * _CH


def _score_body(x_ref, w_ref, out_ref):
    x = x_ref[...]
    w = w_ref[...]
    s = jnp.sum(x * w, axis=0, keepdims=True)
    i = pl.program_id(0)
    col = lax.broadcasted_iota(jnp.int32, (1, _COLS_A), 1)
    s = jnp.where((i == 0) & (col == _PAD), 0.0, s)
    out_ref[...] = s.reshape(_COLS_A)


def _scores(table_t, w_col):
    return pl.pallas_call(
        _score_body,
        grid=(_GRID_A,),
        in_specs=[
            pl.BlockSpec((_EMB, _COLS_A), lambda i: (0, i)),
            pl.BlockSpec((_EMB, 1), lambda i: (0, 0)),
        ],
        out_specs=pl.BlockSpec((_COLS_A,), lambda i: (i,)),
        out_shape=jax.ShapeDtypeStruct((_VOCAB,), jnp.float32),
    )(table_t, w_col)


def _sc_pool_body(scores_hbm, ids_hbm, bvec_hbm, out_hbm,
                  idx0, idx1, vals0, vals1, out_v, b_v, semg, sema):
    wid = lax.axis_index("s") * _NC + lax.axis_index("c")
    base = wid * _COLS_W
    pltpu.sync_copy(bvec_hbm, b_v)
    bv = b_v[...]
    idx = [idx0, idx1]
    vals = [vals0, vals1]

    def assemble(ci, buf):
        col0 = base + ci * _CH

        def cp_issue(l, c2):
            pltpu.async_copy(ids_hbm.at[l, pl.ds(col0, _CH)],
                             buf.at[pl.ds(l * _CH, _CH)], sema)
            return c2

        def cp_drain(l, c2):
            pltpu.make_async_copy(ids_hbm.at[l, pl.ds(col0, _CH)],
                                  buf.at[pl.ds(l * _CH, _CH)], sema).wait()
            return c2

        lax.fori_loop(0, _L, cp_issue, 0)
        lax.fori_loop(0, _L, cp_drain, 0)

    assemble(0, idx[0])
    pltpu.async_copy(scores_hbm.at[idx[0]], vals[0], semg)
    assemble(1, idx[1])
    for ci in range(_NCH):
        cur = ci % 2
        pltpu.make_async_copy(scores_hbm.at[idx[cur]], vals[cur], semg).wait()
        if ci + 1 < _NCH:
            pltpu.async_copy(scores_hbm.at[idx[1 - cur]], vals[1 - cur], semg)
        if ci + 2 < _NCH:
            assemble(ci + 2, idx[cur])

        for k in range(_CH // 16):
            def red(l, acc):
                return acc + vals[cur][pl.ds(l * _CH + k * 16, 16)]
            acc = lax.fori_loop(0, _L, red, jnp.zeros((16,), jnp.float32))
            z = acc + bv
            y = 1.0 / (1.0 + jnp.exp(-z))
            out_v[pl.ds(ci * _CH + k * 16, 16)] = y

    pltpu.sync_copy(out_v, out_hbm.at[pl.ds(base, _COLS_W)])


def _sc_pool(scores, ids_t, bvec):
    mesh = plsc.VectorSubcoreMesh(core_axis_name="c", subcore_axis_name="s")
    f = pl.kernel(
        _sc_pool_body,
        out_type=jax.ShapeDtypeStruct((_B,), jnp.float32),
        mesh=mesh,
        scratch_types=[
            pltpu.VMEM((_CHW,), jnp.int32),
            pltpu.VMEM((_CHW,), jnp.int32),
            pltpu.VMEM((_CHW,), jnp.float32),
            pltpu.VMEM((_CHW,), jnp.float32),
            pltpu.VMEM((_COLS_W,), jnp.float32),
            pltpu.VMEM((16,), jnp.float32),
            pltpu.SemaphoreType.DMA,
            pltpu.SemaphoreType.DMA,
        ],
    )
    return f(scores, ids_t, bvec)


def kernel(ids, table, W, b):
    table_t = table.astype(jnp.float32).T
    ids_t = ids.astype(jnp.int32).T
    w_col = W.astype(jnp.float32).reshape(_EMB, 1) * (1.0 / _L)
    scores = _scores(table_t, w_col)
    bvec = jnp.broadcast_to(b.astype(jnp.float32), (16,))
    out_flat = _sc_pool(scores, ids_t, bvec)
    return out_flat.reshape(_B, 1)


# speedup vs baseline: 4.3773x; 1.0347x over previous
import functools

import jax
import jax.numpy as jnp
from jax import lax
from jax.experimental import pallas as pl
from jax.experimental.pallas import tpu as pltpu
from jax.experimental.pallas import tpu_sc as plsc

_VOCAB = 1000000
_EMB = 32
_B = 16384
_L = 200
_PAD = 0

_COLS_A = 65536
_GRID_A = -(-_VOCAB // _COLS_A)

_NC, _NS = 2, 16
_NW = _NC * _NS
_COLS_W = _B // _NW
_CH = 128
_NCH = _COLS_W // _CH
_CHW = _L * _CH


def _score_body(x_ref, w_ref, out_ref):
    x = x_ref[...]
    w = w_ref[...]
    s = jnp.sum(x * w, axis=0, keepdims=True)
    i = pl.program_id(0)
    col = lax.broadcasted_iota(jnp.int32, (1, _COLS_A), 1)
    s = jnp.where((i == 0) & (col == _PAD), 0.0, s)
    out_ref[...] = s.reshape(_COLS_A)


def _scores(table_t, w_col):
    return pl.pallas_call(
        _score_body,
        grid=(_GRID_A,),
        in_specs=[
            pl.BlockSpec((_EMB, _COLS_A), lambda i: (0, i)),
            pl.BlockSpec((_EMB, 1), lambda i: (0, 0)),
        ],
        out_specs=pl.BlockSpec((_COLS_A,), lambda i: (i,)),
        out_shape=jax.ShapeDtypeStruct((_VOCAB,), jnp.float32),
    )(table_t, w_col)


def _sc_pool_body(scores_hbm, ids_hbm, bvec_hbm, out_hbm,
                  idx0, idx1, vals0, vals1, out_v, b_v, semg, sema):
    wid = lax.axis_index("s") * _NC + lax.axis_index("c")
    base = wid * _COLS_W
    pltpu.sync_copy(bvec_hbm, b_v)
    bv = b_v[...]
    idx = [idx0, idx1]
    vals = [vals0, vals1]

    def assemble(ci, buf):
        col0 = base + ci * _CH

        def cp_issue(l, c2):
            pltpu.async_copy(ids_hbm.at[l, pl.ds(col0, _CH)],
                             buf.at[pl.ds(l * _CH, _CH)], sema)
            return c2

        def cp_drain(l, c2):
            pltpu.make_async_copy(ids_hbm.at[l, pl.ds(col0, _CH)],
                                  buf.at[pl.ds(l * _CH, _CH)], sema).wait()
            return c2

        lax.fori_loop(0, _L, cp_issue, 0)
        lax.fori_loop(0, _L, cp_drain, 0)

    assemble(0, idx[0])
    pltpu.async_copy(scores_hbm.at[idx[0]], vals[0], semg)
    assemble(1, idx[1])
    for ci in range(_NCH):
        cur = ci % 2
        pltpu.make_async_copy(scores_hbm.at[idx[cur]], vals[cur], semg).wait()
        if ci + 1 < _NCH:
            pltpu.async_copy(scores_hbm.at[idx[1 - cur]], vals[1 - cur], semg)
        if ci + 2 < _NCH:
            assemble(ci + 2, idx[cur])

        for k in range(_CH // 16):
            def red(l, acc):
                return acc + vals[cur][pl.ds(l * _CH + k * 16, 16)]
            acc = lax.fori_loop(0, _L, red, jnp.zeros((16,), jnp.float32))
            z = acc + bv
            y = 1.0 / (1.0 + jnp.exp(-z))
            out_v[pl.ds(ci * _CH + k * 16, 16)] = y

    pltpu.sync_copy(out_v, out_hbm.at[pl.ds(base, _COLS_W)])


def _sc_pool(scores, ids_t, bvec):
    mesh = plsc.VectorSubcoreMesh(core_axis_name="c", subcore_axis_name="s")
    f = pl.kernel(
        _sc_pool_body,
        out_type=jax.ShapeDtypeStruct((_B,), jnp.float32),
        mesh=mesh,
        scratch_types=[
            pltpu.VMEM((_CHW,), jnp.int32),
            pltpu.VMEM((_CHW,), jnp.int32),
            pltpu.VMEM((_CHW,), jnp.float32),
            pltpu.VMEM((_CHW,), jnp.float32),
            pltpu.VMEM((_COLS_W,), jnp.float32),
            pltpu.VMEM((16,), jnp.float32),
            pltpu.SemaphoreType.DMA,
            pltpu.SemaphoreType.DMA,
        ],
    )
    return f(scores, ids_t, bvec)


def kernel(ids, table, W, b):
    table_t = table.astype(jnp.float32).T
    ids_t = ids.astype(jnp.int32).T
    w_col = W.astype(jnp.float32).reshape(_EMB, 1) * (1.0 / _L)
    scores = _scores(table_t, w_col)
    bvec = jnp.broadcast_to(b.astype(jnp.float32), (16,))
    out_flat = _sc_pool(scores, ids_t, bvec)
    return out_flat.reshape(_B, 1)
